# baseline (speedup 1.0000x reference)
import jax
import jax.numpy as jnp
from jax import lax
from jax.experimental import pallas as pl
from jax.experimental.pallas import tpu as pltpu
from jax.experimental.pallas import tpu_sc as plsc

NC, NS, L = 2, 16, 16
NW = NC * NS
CHUNK = 128
NPAD = 102400
RPT = NPAD // NS
ACCR = 51264
APT = ACCR // NS
DUMMY = 51200


def _mesh():
    return plsc.VectorSubcoreMesh(core_axis_name="c", subcore_axis_name="s")


KH1 = 8
KH4 = 2
GRAN = CHUNK * NW * KH1


def _sc_count(dst2d, ones_hbm, z8_hbm):
    nch = dst2d.shape[0]
    K = KH1
    niter = nch // (K * NW)

    def body(dst_hbm, ones_ref, z_ref, out_hbm, dbuf, ones, cacc):
        cid = lax.axis_index("c")
        sid = lax.axis_index("s")
        wid = sid * NC + cid
        base0 = sid * RPT
        pltpu.sync_copy(z_ref.at[pl.ds(0, RPT)], cacc.at[pl.ds(base0, RPT)])
        pltpu.sync_copy(ones_ref, ones)
        plsc.subcore_barrier()

        def step(i, c):
            g = i * NW + wid
            pltpu.sync_copy(dst_hbm.at[pl.ds(g * K, K)], dbuf)
            for k in range(K):
                pltpu.sync_copy(ones, cacc.at[dbuf.at[k]], add=True)
            return c

        lax.fori_loop(0, niter, step, 0)
        plsc.subcore_barrier()
        pltpu.sync_copy(cacc.at[pl.ds(base0, RPT)],
                        out_hbm.at[cid, pl.ds(base0, RPT)])

    k = pl.kernel(
        body,
        out_type=jax.ShapeDtypeStruct((NC, NPAD, 8), jnp.float32),
        mesh=_mesh(),
        compiler_params=pltpu.CompilerParams(use_tc_tiling_on_sc=False),
        scratch_types=[
            pltpu.VMEM((K, CHUNK), jnp.int32),
            pltpu.VMEM((CHUNK, 8), jnp.float32),
            pltpu.VMEM_SHARED((NPAD, 8), jnp.float32),
        ],
    )
    return k(dst2d, ones_hbm, z8_hbm)


def _sc_pass_h1(h_tab, src2d, dst2d, z_hbm, half):
    nch = src2d.shape[0]
    K = KH1
    niter = nch // (K * NS)

    def body(h_hbm, src_hbm, dst_hbm, z_ref, out_hbm,
             sbuf, dbuf, rbuf, rows, acc, sem0, sem1):
        cid = lax.axis_index("c")
        sid = lax.axis_index("s")
        base0 = sid * APT
        off = cid * half
        pltpu.sync_copy(z_ref.at[pl.ds(0, APT)], acc.at[pl.ds(base0, APT)])
        plsc.subcore_barrier()
        sems = (sem0, sem1)

        def load_idx(b, g):
            pltpu.sync_copy(src_hbm.at[pl.ds(g * K, K)], sbuf.at[b])
            pltpu.sync_copy(dst_hbm.at[pl.ds(g * K, K)], dbuf.at[b])

        def remap(b):
            for k in range(K):
                for v in range(CHUNK // L):
                    d = dbuf[b, k, pl.ds(v * L, L)] - off
                    ok = (d >= 0) & (d < half)
                    rbuf[b, k, pl.ds(v * L, L)] = jnp.where(ok, d, DUMMY)

        def fire(b):
            for k in range(K):
                pltpu.async_copy(h_hbm.at[sbuf.at[b].at[k]],
                                 rows.at[b].at[k], sems[b])

        def drain(b):
            for k in range(K):
                pltpu.make_async_copy(h_hbm.at[sbuf.at[b].at[k]],
                                      rows.at[b].at[k], sems[b]).wait()

        def process(b):
            for k in range(K):
                pltpu.sync_copy(rows.at[b].at[k], acc.at[rbuf.at[b].at[k]],
                                add=True)

        load_idx(0, sid)
        fire(0)
        remap(0)

        def two(jp, c):
            for b in (0, 1):
                j = jp * 2 + b
                load_idx(1 - b, (j + 1) * NS + sid)
                fire(1 - b)
                remap(1 - b)
                drain(b)
                process(b)
            return c

        pairs = (niter - 1) // 2
        lax.fori_loop(0, pairs, two, 0)
        for j in range(2 * pairs, niter):
            b = j % 2
            if j + 1 < niter:
                load_idx(1 - b, (j + 1) * NS + sid)
                fire(1 - b)
                remap(1 - b)
            drain(b)
            process(b)

        plsc.subcore_barrier()
        pltpu.sync_copy(acc.at[pl.ds(base0, APT)],
                        out_hbm.at[cid, pl.ds(base0, APT)])

    k = pl.kernel(
        body,
        out_type=jax.ShapeDtypeStruct((NC, ACCR, 16), jnp.float32),
        mesh=_mesh(),
        compiler_params=pltpu.CompilerParams(use_tc_tiling_on_sc=False),
        scratch_types=[
            pltpu.VMEM((2, K, CHUNK), jnp.int32),
            pltpu.VMEM((2, K, CHUNK), jnp.int32),
            pltpu.VMEM((2, K, CHUNK), jnp.int32),
            pltpu.VMEM((2, K, CHUNK, 16), jnp.float32),
            pltpu.VMEM_SHARED((ACCR, 16), jnp.float32),
            pltpu.SemaphoreType.DMA,
            pltpu.SemaphoreType.DMA,
        ],
    )
    return k(h_tab, src2d, dst2d, z_hbm)


def _sc_pass_h4(p_tab, y_tab, src2d, dst2d, dstg2d, ctile, z_hbm, half):
    nch = src2d.shape[0]
    K = KH4
    niter = nch // (K * NS)

    def body(p_hbm, y_hbm, src_hbm, dst_hbm, dstg_hbm, c_hbm, z_ref, out_hbm,
             sbuf, dbuf, dgbuf, pa, pb, yb, msg, cbuf, acc, sem0, sem1):
        cid = lax.axis_index("c")
        sid = lax.axis_index("s")
        base0 = sid * APT
        off = cid * half
        pltpu.sync_copy(z_ref.at[pl.ds(0, APT)], acc.at[pl.ds(base0, APT)])
        pltpu.sync_copy(c_hbm, cbuf)
        plsc.subcore_barrier()
        sems = (sem0, sem1)
        cvec = cbuf[...]

        def load_idx(b, g):
            pltpu.sync_copy(src_hbm.at[pl.ds(g * K, K)], sbuf.at[b])
            pltpu.sync_copy(dst_hbm.at[pl.ds(g * K, K)], dbuf.at[b])
            pltpu.sync_copy(dstg_hbm.at[pl.ds(g * K, K)], dgbuf.at[b])

        def remap(b):
            for k in range(K):
                for v in range(CHUNK // L):
                    d = dbuf[b, k, pl.ds(v * L, L)] - off
                    ok = (d >= 0) & (d < half)
                    dbuf[b, k, pl.ds(v * L, L)] = jnp.where(ok, d, DUMMY)

        def fire(b):
            for k in range(K):
                pltpu.async_copy(p_hbm.at[sbuf.at[b].at[k]],
                                 pa.at[b].at[k], sems[b])
                pltpu.async_copy(p_hbm.at[dgbuf.at[b].at[k]],
                                 pb.at[b].at[k], sems[b])
                pltpu.async_copy(y_hbm.at[sbuf.at[b].at[k]],
                                 yb.at[b].at[k], sems[b])

        def drain(b):
            for k in range(K):
                pltpu.make_async_copy(p_hbm.at[sbuf.at[b].at[k]],
                                      pa.at[b].at[k], sems[b]).wait()
                pltpu.make_async_copy(p_hbm.at[dgbuf.at[b].at[k]],
                                      pb.at[b].at[k], sems[b]).wait()
                pltpu.make_async_copy(y_hbm.at[sbuf.at[b].at[k]],
                                      yb.at[b].at[k], sems[b]).wait()

        def process(b):
            for k in range(K):
                pak = pa.at[b].at[k]
                pbk = pb.at[b].at[k]
                ybk = yb.at[b].at[k]
                msgk = msg.at[k]

                def edge(e, cc):
                    ex = jnp.exp(pak[e, :] - pbk[e, :] + cvec)
                    s = ex[0] + ex[1] + ex[2] + ex[3]
                    m = None
                    for h in range(4):
                        yv = ybk[e, pl.ds(16 * h, 16)]
                        term = ex[h] * yv
                        m = term if m is None else m + term
                    msgk[e, :] = m / s
                    return cc

                lax.fori_loop(0, CHUNK, edge, 0, unroll=8)
            for k in range(K):
                pltpu.sync_copy(msg.at[k], acc.at[dbuf.at[b].at[k]], add=True)

        load_idx(0, sid)
        fire(0)
        remap(0)

        def two(jp, c):
            for b in (0, 1):
                j = jp * 2 + b
                load_idx(1 - b, (j + 1) * NS + sid)
                fire(1 - b)
                remap(1 - b)
                drain(b)
                process(b)
            return c

        pairs = (niter - 1) // 2
        lax.fori_loop(0, pairs, two, 0)
        for j in range(2 * pairs, niter):
            b = j % 2
            if j + 1 < niter:
                load_idx(1 - b, (j + 1) * NS + sid)
                fire(1 - b)
                remap(1 - b)
            drain(b)
            process(b)

        plsc.subcore_barrier()
        pltpu.sync_copy(acc.at[pl.ds(base0, APT)],
                        out_hbm.at[cid, pl.ds(base0, APT)])

    k = pl.kernel(
        body,
        out_type=jax.ShapeDtypeStruct((NC, ACCR, 16), jnp.float32),
        mesh=_mesh(),
        compiler_params=pltpu.CompilerParams(use_tc_tiling_on_sc=False),
        scratch_types=[
            pltpu.VMEM((2, K, CHUNK), jnp.int32),
            pltpu.VMEM((2, K, CHUNK), jnp.int32),
            pltpu.VMEM((2, K, CHUNK), jnp.int32),
            pltpu.VMEM((2, K, CHUNK, 16), jnp.float32),
            pltpu.VMEM((2, K, CHUNK, 16), jnp.float32),
            pltpu.VMEM((2, K, CHUNK, 64), jnp.float32),
            pltpu.VMEM((K, CHUNK, 16), jnp.float32),
            pltpu.VMEM((16,), jnp.float32),
            pltpu.VMEM_SHARED((ACCR, 16), jnp.float32),
            pltpu.SemaphoreType.DMA,
            pltpu.SemaphoreType.DMA,
        ],
    )
    return k(p_tab, y_tab, src2d, dst2d, dstg2d, ctile, z_hbm)


RB = 2000


def _rows(d):
    return pl.BlockSpec((RB, d), lambda i: (i, 0))


def _full(shape):
    return pl.BlockSpec(shape, lambda i: tuple(0 for _ in shape))


def _tc_call(fn, n, ins, in_specs, out_shapes, out_specs):
    return pl.pallas_call(
        fn,
        grid=(n // RB,),
        in_specs=in_specs,
        out_specs=out_specs,
        out_shape=out_shapes,
    )(*ins)


def _wc(W, c, heads, out_ch):
    q = jax.nn.softmax(c.reshape(heads))
    return (W.reshape(W.shape[0], heads, out_ch) * q[None, :, None]).sum(axis=1)


def kernel(x, edge_index, W2, u2, c2, b2, W3, u3, c3, b3, W4, u4, c4, b4,
           W5, u5, c5, b5, W6, u6, c6, b6, lin1_w, lin1_b, lin2_w, lin2_b,
           out_w, out_b):
    n = x.shape[0]
    half = n // 2
    e = edge_index.shape[1]
    epad = ((e + GRAN - 1) // GRAN) * GRAN
    src = jnp.concatenate(
        [edge_index[0], jnp.zeros((epad - e,), edge_index.dtype)])
    dst = jnp.concatenate(
        [edge_index[1], jnp.full((epad - e,), n, edge_index.dtype)])
    dst_g = jnp.concatenate(
        [edge_index[1], jnp.zeros((epad - e,), edge_index.dtype)])
    src = src.reshape(epad // CHUNK, CHUNK)
    dst = dst.reshape(epad // CHUNK, CHUNK)
    dst_g = dst_g.reshape(epad // CHUNK, CHUNK)
    z_hbm = jnp.zeros((RPT, 16), jnp.float32)
    z8_hbm = jnp.zeros((RPT, 8), jnp.float32)
    ones_hbm = jnp.ones((CHUNK, 8), jnp.float32)

    def halves(parts):
        return jnp.concatenate([parts[0, :half], parts[1, :half]], axis=0)

    cnt_parts = _sc_count(dst, ones_hbm, z8_hbm)
    cnt0 = cnt_parts[0, :n, 0].reshape(n, 1)
    cnt1 = cnt_parts[1, :n, 0].reshape(n, 1)

    def tc1(x_ref, W_ref, u_ref, Wc_ref, y_ref, p_ref, s_ref):
        h0 = jnp.maximum(x_ref[...], 0.0)
        y_ref[...] = jnp.dot(h0, W_ref[...], preferred_element_type=jnp.float32)
        p_ref[...] = jnp.dot(h0, u_ref[...], preferred_element_type=jnp.float32)
        s_ref[...] = jnp.dot(h0, Wc_ref[...], preferred_element_type=jnp.float32)

    W2c = _wc(W2, c2, 4, 16)
    u2p = jnp.pad(u2, ((0, 0), (0, 12)))
    y2, p2, self2 = _tc_call(
        tc1, n, [x, W2, u2p, W2c],
        [_rows(16), _full((16, 64)), _full((16, 16)), _full((16, 16))],
        [jax.ShapeDtypeStruct((n, 64), jnp.float32),
         jax.ShapeDtypeStruct((n, 16), jnp.float32),
         jax.ShapeDtypeStruct((n, 16), jnp.float32)],
        [_rows(64), _rows(16), _rows(16)],
    )

    a2 = halves(_sc_pass_h4(p2, y2, src, dst, dst_g, jnp.pad(c2, (0, 12)),
                            z_hbm, half))

    def tc2(aa, sf, ca, cb, b_ref, W_ref, u_ref, Wc_ref,
            y_ref, p_ref, s_ref, inv_ref):
        cnt = ca[...] + cb[...] + 1.0
        inv = 1.0 / cnt
        inv_ref[...] = inv
        h1 = jnp.maximum((aa[...] + sf[...]) * inv + b_ref[...], 0.0)
        y_ref[...] = jnp.dot(h1, W_ref[...], preferred_element_type=jnp.float32)
        p_ref[...] = jnp.dot(h1, u_ref[...], preferred_element_type=jnp.float32)
        s_ref[...] = jnp.dot(h1, Wc_ref[...], preferred_element_type=jnp.float32)

    W3c = _wc(W3, c3, 4, 16)
    u3p = jnp.pad(u3, ((0, 0), (0, 12)))
    y3, p3, self3, inv = _tc_call(
        tc2, n, [a2, self2, cnt0, cnt1, b2.reshape(1, 16), W3, u3p, W3c],
        [_rows(16), _rows(16), _rows(1), _rows(1),
         _full((1, 16)), _full((16, 64)), _full((16, 16)), _full((16, 16))],
        [jax.ShapeDtypeStruct((n, 64), jnp.float32),
         jax.ShapeDtypeStruct((n, 16), jnp.float32),
         jax.ShapeDtypeStruct((n, 16), jnp.float32),
         jax.ShapeDtypeStruct((n, 1), jnp.float32)],
        [_rows(64), _rows(16), _rows(16), _rows(1)],
    )

    a3 = halves(_sc_pass_h4(p3, y3, src, dst, dst_g, jnp.pad(c3, (0, 12)),
                            z_hbm, half))

    def tc3(aa, sf, inv_ref, b_ref, h_ref):
        h_ref[...] = jnp.maximum(
            (aa[...] + sf[...]) * inv_ref[...] + b_ref[...], 0.0)

    h2 = _tc_call(
        tc3, n, [a3, self3, inv, b3.reshape(1, 16)],
        [_rows(16), _rows(16), _rows(1), _full((1, 16))],
        jax.ShapeDtypeStruct((n, 16), jnp.float32),
        _rows(16),
    )

    s4 = halves(_sc_pass_h1(h2, src, dst, z_hbm, half))

    def tc4(aa, hp, inv_ref, W_ref, b_ref, h_ref):
        agg = (aa[...] + hp[...]) * inv_ref[...]
        h_ref[...] = jnp.maximum(
            jnp.dot(agg, W_ref[...], preferred_element_type=jnp.float32)
            + b_ref[...], 0.0)

    h3 = _tc_call(
        tc4, n, [s4, h2, inv, W4, b4.reshape(1, 16)],
        [_rows(16), _rows(16), _rows(1), _full((16, 16)), _full((1, 16))],
        jax.ShapeDtypeStruct((n, 16), jnp.float32),
        _rows(16),
    )

    s5 = halves(_sc_pass_h1(h3, src, dst, z_hbm, half))

    def tc5(aa, hp, inv_ref, W_ref, b_ref, h_ref):
        agg = (aa[...] + hp[...]) * inv_ref[...]
        h_ref[...] = jnp.maximum(
            jnp.dot(agg, W_ref[...], preferred_element_type=jnp.float32)
            + b_ref[...], 0.0)

    h4 = _tc_call(
        tc5, n, [s5, h3, inv, W5, b5.reshape(1, 32)],
        [_rows(16), _rows(16), _rows(1), _full((16, 32)), _full((1, 32))],
        jax.ShapeDtypeStruct((n, 32), jnp.float32),
        _rows(32),
    )

    h4a = h4[:, :16]
    h4b = h4[:, 16:]
    s6a = halves(_sc_pass_h1(h4a, src, dst, z_hbm, half))
    s6b = halves(_sc_pass_h1(h4b, src, dst, z_hbm, half))

    def tc6(sa0, sa1, ha, hb, inv_ref, W_ref, b_ref,
            l1w, l1b, l2w, l2b, ow, ob, o_ref):
        agg0 = (sa0[...] + ha[...]) * inv_ref[...]
        agg1 = (sa1[...] + hb[...]) * inv_ref[...]
        agg = jnp.concatenate([agg0, agg1], axis=1)
        h5 = jnp.maximum(
            jnp.dot(agg, W_ref[...], preferred_element_type=jnp.float32)
            + b_ref[...], 0.0)
        h6 = jnp.maximum(
            jnp.dot(h5, l1w[...], preferred_element_type=jnp.float32)
            + l1b[...], 0.0)
        h7 = jnp.maximum(
            jnp.dot(h6, l2w[...], preferred_element_type=jnp.float32)
            + l2b[...], 0.0)
        o = jnp.dot(h7, ow[...], preferred_element_type=jnp.float32) + ob[...]
        o_ref[...] = jax.nn.sigmoid(o)

    out = _tc_call(
        tc6, n,
        [s6a, s6b, h4a, h4b, inv,
         W6, b6.reshape(1, 64), lin1_w, lin1_b.reshape(1, 16),
         lin2_w, lin2_b.reshape(1, 4), out_w, out_b.reshape(1, 1)],
        [_rows(16), _rows(16), _rows(16), _rows(16), _rows(1),
         _full((32, 64)), _full((1, 64)), _full((64, 16)), _full((1, 16)),
         _full((16, 4)), _full((1, 4)), _full((4, 1)), _full((1, 1))],
        jax.ShapeDtypeStruct((n, 1), jnp.float32),
        _rows(1),
    )
    return out
